# Initial kernel scaffold; baseline (speedup 1.0000x reference)
import jax
import jax.numpy as jnp
from jax.experimental import pallas as pl


def kernel(x, table):
    raise NotImplementedError("write your pallas kernel here")


# speedup vs baseline: 3.3705x; 3.3705x over previous
import functools

import jax
import jax.numpy as jnp
from jax import lax
from jax.experimental import pallas as pl
from jax.experimental.pallas import tpu as pltpu
from jax.experimental.pallas import tpu_sc as plsc

_BATCH = 16384
_FIELDS = 26
_DIM = 128

_NC = 2
_NS = 16
_NW = _NC * _NS
_C = 128
_B = _BATCH * _FIELDS
_BPW = _B // _NW
_NCHUNK = _BPW // _C


def _sc_gather(idx_hbm, table_hbm, out_hbm, idx_v, buf0, buf1,
               gsem0, gsem1, osem0, osem1):
    wid = lax.axis_index("s") * _NC + lax.axis_index("c")
    base = wid * _BPW

    pltpu.sync_copy(idx_hbm.at[wid], idx_v)

    bufs = (buf0, buf1)
    gsems = (gsem0, gsem1)
    osems = (osem0, osem1)

    pltpu.async_copy(table_hbm.at[idx_v.at[0]], buf0, gsem0)
    pltpu.async_copy(table_hbm.at[idx_v.at[1]], buf1, gsem1)

    @pl.loop(0, _NCHUNK, step=2)
    def _pair(j0):
        for b in range(2):
            j = j0 + b
            buf = bufs[b]
            dst = out_hbm.at[pl.ds(base + j * _C, _C)]
            pltpu.make_async_copy(table_hbm.at[idx_v.at[j]], buf,
                                  gsems[b]).wait()
            pltpu.async_copy(buf, dst, osems[b])

            @pl.when(j < _NCHUNK - 2)
            def _():
                pltpu.make_async_copy(buf, dst, osems[b]).wait()
                pltpu.async_copy(table_hbm.at[idx_v.at[j + 2]], buf, gsems[b])

    pltpu.make_async_copy(buf0, out_hbm.at[pl.ds(base, _C)], osem0).wait()
    pltpu.make_async_copy(buf1, out_hbm.at[pl.ds(base, _C)], osem1).wait()


_gather_call = functools.partial(
    pl.kernel,
    out_type=jax.ShapeDtypeStruct((_B, _DIM), jnp.float32),
    mesh=plsc.VectorSubcoreMesh(core_axis_name="c", subcore_axis_name="s",
                                num_cores=_NC, num_subcores=_NS),
    scratch_types=[
        pltpu.VMEM((_NCHUNK, _C), jnp.int32),
        pltpu.VMEM((_C, _DIM), jnp.float32),
        pltpu.VMEM((_C, _DIM), jnp.float32),
        pltpu.SemaphoreType.DMA,
        pltpu.SemaphoreType.DMA,
        pltpu.SemaphoreType.DMA,
        pltpu.SemaphoreType.DMA,
    ],
)(_sc_gather)


@jax.jit
def kernel(x, table):
    idx = x.reshape(_NW, _NCHUNK, _C).astype(jnp.int32)
    out = _gather_call(idx, table)
    return out.reshape(_BATCH, _FIELDS, _DIM)


# speedup vs baseline: 3.3852x; 1.0044x over previous
import functools

import jax
import jax.numpy as jnp
from jax import lax
from jax.experimental import pallas as pl
from jax.experimental.pallas import tpu as pltpu
from jax.experimental.pallas import tpu_sc as plsc

_BATCH = 16384
_FIELDS = 26
_DIM = 128

_NC = 2
_NS = 16
_NW = _NC * _NS
_C = 128
_B = _BATCH * _FIELDS
_BPW = _B // _NW
_NCHUNK = _BPW // _C


_NBUF = 4


def _sc_gather(idx_hbm, table_hbm, out_hbm, idx_v, *bufs_and_sems):
    bufs = bufs_and_sems[:_NBUF]
    gsems = bufs_and_sems[_NBUF:2 * _NBUF]
    osems = bufs_and_sems[2 * _NBUF:3 * _NBUF]

    wid = lax.axis_index("s") * _NC + lax.axis_index("c")
    base = wid * _BPW

    pltpu.sync_copy(idx_hbm.at[wid], idx_v)

    for b in range(_NBUF):
        pltpu.async_copy(table_hbm.at[idx_v.at[b]], bufs[b], gsems[b])

    @pl.loop(0, _NCHUNK, step=_NBUF)
    def _group(j0):
        for b in range(_NBUF):
            j = j0 + b
            buf = bufs[b]
            dst = out_hbm.at[pl.ds(base + j * _C, _C)]
            pltpu.make_async_copy(table_hbm.at[idx_v.at[j]], buf,
                                  gsems[b]).wait()
            pltpu.async_copy(buf, dst, osems[b])

            @pl.when(j < _NCHUNK - _NBUF)
            def _():
                pltpu.make_async_copy(buf, dst, osems[b]).wait()
                pltpu.async_copy(table_hbm.at[idx_v.at[j + _NBUF]], buf,
                                 gsems[b])

    for b in range(_NBUF):
        pltpu.make_async_copy(bufs[b], out_hbm.at[pl.ds(base, _C)],
                              osems[b]).wait()


_gather_call = functools.partial(
    pl.kernel,
    out_type=jax.ShapeDtypeStruct((_B, _DIM), jnp.float32),
    mesh=plsc.VectorSubcoreMesh(core_axis_name="c", subcore_axis_name="s",
                                num_cores=_NC, num_subcores=_NS),
    scratch_types=(
        [pltpu.VMEM((_NCHUNK, _C), jnp.int32)]
        + [pltpu.VMEM((_C, _DIM), jnp.float32)] * _NBUF
        + [pltpu.SemaphoreType.DMA] * (2 * _NBUF)
    ),
)(_sc_gather)


@jax.jit
def kernel(x, table):
    idx = x.reshape(_NW, _NCHUNK, _C).astype(jnp.int32)
    out = _gather_call(idx, table)
    return out.reshape(_BATCH, _FIELDS, _DIM)
